# baseline (speedup 1.0000x reference)
import jax
import jax.numpy as jnp
from jax.experimental import pallas as pl
from jax.experimental.pallas import tpu as pltpu

DIM = 1024
RED = 256
E = 4
HID = 32
EH = E * HID
TB = 1024
NT = 3


def _body(x_ref, t_ref, wr_hbm, g1_hbm, b1_hbm, g2_hbm, b2_hbm, wg_hbm,
          we1_hbm, be1_hbm, we2_hbm, be2_hbm, ms_hbm, ti_ref,
          ox_ref, ot_ref, pxpt_ref, aux_ref,
          wr_raw, wbx_ref, wbt_ref, c2_ref, g2_v, b2_v, wg_v, we1_v, be1_v,
          w1g_ref, b1g_ref, s_ref, w2_ref, be2_v, ms_v, sem,
          imp_ref, load_ref):
    i = pl.program_id(0)
    nsteps = pl.num_programs(0)
    ti = ti_ref[0, 0]

    @pl.when(i == 0)
    def _prep():
        cps = [pltpu.make_async_copy(wr_hbm, wr_raw.at[0:RED, :], sem),
               pltpu.make_async_copy(g1_hbm, wr_raw.at[RED:RED + 1, :], sem),
               pltpu.make_async_copy(b1_hbm, wr_raw.at[RED + 1:RED + 2, :],
                                     sem),
               pltpu.make_async_copy(g2_hbm, g2_v, sem),
               pltpu.make_async_copy(b2_hbm, b2_v, sem),
               pltpu.make_async_copy(wg_hbm, wg_v, sem),
               pltpu.make_async_copy(we1_hbm, we1_v, sem),
               pltpu.make_async_copy(be1_hbm, be1_v, sem),
               pltpu.make_async_copy(we2_hbm, w2_ref, sem),
               pltpu.make_async_copy(be2_hbm, be2_v, sem),
               pltpu.make_async_copy(ms_hbm, ms_v, sem)]
        for cp in cps:
            cp.start()
        for cp in cps:
            cp.wait()

        wr = wr_raw[0:RED, :]
        g1 = wr_raw[RED:RED + 1, :]
        wbx_ref[...] = (wr[:, :DIM] * g1[:, :DIM]).astype(jnp.bfloat16)
        wbt_ref[...] = (wr[:, DIM:] * g1[:, DIM:]).astype(jnp.bfloat16)
        c2_ref[...] = jax.lax.dot_general(
            wr_raw[RED + 1:RED + 2, :], wr, (((1,), (1,)), ((), ())),
            preferred_element_type=jnp.float32)
        we1f = jnp.concatenate([we1_v[e] for e in range(E)], axis=1)
        w1g_ref[...] = jnp.concatenate([we1f, wg_v[ti]],
                                       axis=1).astype(jnp.bfloat16)
        b1g_ref[...] = jnp.concatenate(
            [be1_v[e:e + 1, :] for e in range(E)]
            + [jnp.zeros((1, E), jnp.float32)], axis=1)
        col = jax.lax.broadcasted_iota(jnp.int32, (E, EH), 1) // HID
        row = jax.lax.broadcasted_iota(jnp.int32, (E, EH), 0)
        s_ref[...] = (col == row).astype(jnp.bfloat16)

    xb = x_ref[...]
    tb = t_ref[...]

    m = (jnp.sum(xb, axis=1, keepdims=True)
         + jnp.sum(tb, axis=1, keepdims=True)) * (1.0 / (2 * DIM))
    v = (jnp.sum(xb * xb, axis=1, keepdims=True)
         + jnp.sum(tb * tb, axis=1, keepdims=True)) * (1.0 / (2 * DIM)) - m * m
    rs = 1.0 / jnp.sqrt(v + 1e-5)

    xnb = ((xb - m) * rs).astype(jnp.bfloat16)
    tnb = ((tb - m) * rs).astype(jnp.bfloat16)
    u = (jax.lax.dot_general(xnb, wbx_ref[...], (((1,), (1,)), ((), ())),
                             preferred_element_type=jnp.float32)
         + jax.lax.dot_general(tnb, wbt_ref[...], (((1,), (1,)), ((), ())),
                               preferred_element_type=jnp.float32))
    u = u + c2_ref[...]

    m2 = jnp.mean(u, axis=1, keepdims=True)
    uc = u - m2
    v2 = jnp.mean(uc * uc, axis=1, keepdims=True)
    yf = uc * (1.0 / jnp.sqrt(v2 + 1e-5)) * g2_v[...] + b2_v[...]
    zb = yf.astype(jnp.bfloat16)

    r = jnp.dot(zb, w1g_ref[...], preferred_element_type=jnp.float32) + b1g_ref[...]
    h = jnp.maximum(r[:, :EH], 0.0)
    logits = r[:, EH:EH + E]

    pri = (E - jax.lax.broadcasted_iota(jnp.int32, logits.shape, 1)
           ).astype(jnp.float32)
    m1 = jnp.max(logits, axis=1, keepdims=True)
    w1m = jnp.where(logits == m1, pri, 0.0)
    mask1 = w1m == jnp.max(w1m, axis=1, keepdims=True)
    l2 = jnp.where(mask1, -jnp.inf, logits)
    m2g = jnp.max(l2, axis=1, keepdims=True)
    w2m = jnp.where(l2 == m2g, pri, 0.0)
    mask2 = w2m == jnp.max(w2m, axis=1, keepdims=True)
    e2 = jnp.exp(m2g - m1)
    den = 1.0 + e2
    gates = (jnp.where(mask1, 1.0 / den, 0.0)
             + jnp.where(mask2, e2 / den, 0.0))

    gb = gates.astype(jnp.bfloat16)
    gexp = jnp.dot(gb, s_ref[...], preferred_element_type=jnp.float32)
    ghb = (h * gexp).astype(jnp.bfloat16)
    moe = (jnp.dot(ghb, w2_ref[...].astype(jnp.bfloat16),
                   preferred_element_type=jnp.float32)
           + jnp.dot(gb, be2_v[...].astype(jnp.bfloat16),
                     preferred_element_type=jnp.float32))

    px = jax.nn.sigmoid(jnp.mean(moe[:, :RED // 2], axis=1, keepdims=True))
    pt = jax.nn.sigmoid(jnp.mean(moe[:, RED // 2:], axis=1, keepdims=True))

    ox_ref[...] = px * xb + ms_v[pl.ds(2 * ti, 1), :]
    ot_ref[...] = pt * tb + ms_v[pl.ds(2 * ti + 1, 1), :]
    pxpt_ref[0:1, 0:1] = jnp.reshape(px, (1, 1, TB // 128, 128))
    pxpt_ref[1:2, 0:1] = jnp.reshape(pt, (1, 1, TB // 128, 128))

    imp_b = jnp.sum(gates, axis=0, keepdims=True)
    load_b = jnp.sum((gates > 0.0).astype(jnp.float32), axis=0, keepdims=True)

    @pl.when(i == 0)
    def _init():
        imp_ref[...] = imp_b
        load_ref[...] = load_b

    @pl.when(i > 0)
    def _acc():
        imp_ref[...] += imp_b
        load_ref[...] += load_b

    @pl.when(i == nsteps - 1)
    def _fin():
        def cv2(a):
            mu = jnp.sum(a, axis=1, keepdims=True) * (1.0 / E)
            var = jnp.sum((a - mu) ** 2, axis=1, keepdims=True) * (1.0 / (E - 1))
            return var / (mu * mu + 1e-10)

        aux_ref[...] = (cv2(imp_ref[...]) + cv2(load_ref[...])) * 1e-2


def kernel(x, t, gamma1, beta1, W_red, gamma2, beta2, w_gate, We1, be1, We2,
           be2, modal_shifts, task_index):
    B, N, C = x.shape
    T = B * N
    xf = x.reshape(T, C)
    tf = t.reshape(T, C)
    ti = jnp.asarray(task_index, jnp.int32).reshape(1, 1)

    grid = (T // TB,)
    tok = lambda i: (i, 0)
    fix = lambda i: (0, 0)
    anyspec = pl.BlockSpec(memory_space=pltpu.MemorySpace.HBM)

    out_x, out_t, pxpt, aux = pl.pallas_call(
        _body,
        grid=grid,
        in_specs=[
            pl.BlockSpec((TB, C), tok),
            pl.BlockSpec((TB, C), tok),
            anyspec,
            anyspec,
            anyspec,
            anyspec,
            anyspec,
            anyspec,
            anyspec,
            anyspec,
            anyspec,
            anyspec,
            anyspec,
            pl.BlockSpec(memory_space=pltpu.SMEM),
        ],
        out_specs=[
            pl.BlockSpec((TB, C), tok),
            pl.BlockSpec((TB, C), tok),
            pl.BlockSpec((2, 1, TB // 128, 128), lambda i: (0, i, 0, 0)),
            pl.BlockSpec((1, 1), fix),
        ],
        out_shape=[
            jax.ShapeDtypeStruct((T, C), jnp.float32),
            jax.ShapeDtypeStruct((T, C), jnp.float32),
            jax.ShapeDtypeStruct((2, T // TB, TB // 128, 128), jnp.float32),
            jax.ShapeDtypeStruct((1, 1), jnp.float32),
        ],
        scratch_shapes=[
            pltpu.VMEM((RED + 2, 2 * DIM), jnp.float32),
            pltpu.VMEM((RED, DIM), jnp.bfloat16),
            pltpu.VMEM((RED, DIM), jnp.bfloat16),
            pltpu.VMEM((1, RED), jnp.float32),
            pltpu.VMEM((1, RED), jnp.float32),
            pltpu.VMEM((1, RED), jnp.float32),
            pltpu.VMEM((NT, RED, E), jnp.float32),
            pltpu.VMEM((E, RED, HID), jnp.float32),
            pltpu.VMEM((E, HID), jnp.float32),
            pltpu.VMEM((RED, EH + E), jnp.bfloat16),
            pltpu.VMEM((1, EH + E), jnp.float32),
            pltpu.VMEM((E, EH), jnp.bfloat16),
            pltpu.VMEM((EH, RED), jnp.float32),
            pltpu.VMEM((E, RED), jnp.float32),
            pltpu.VMEM((2 * NT, DIM), jnp.float32),
            pltpu.SemaphoreType.DMA,
            pltpu.VMEM((1, E), jnp.float32),
            pltpu.VMEM((1, E), jnp.float32),
        ],
        compiler_params=pltpu.CompilerParams(
            dimension_semantics=("arbitrary",),
        ),
    )(xf, tf, W_red, gamma1[None, :], beta1[None, :], gamma2[None, :],
      beta2[None, :], w_gate, We1, be1, We2.reshape(EH, RED), be2,
      modal_shifts, ti)

    return (out_x.reshape(B, N, C), out_t.reshape(B, N, C),
            pxpt[0].reshape(B, N, 1), pxpt[1].reshape(B, N, 1),
            aux.reshape(()))


# speedup vs baseline: 1.0222x; 1.0222x over previous
import jax
import jax.numpy as jnp
from jax.experimental import pallas as pl
from jax.experimental.pallas import tpu as pltpu

DIM = 1024
RED = 256
E = 4
HID = 32
EH = E * HID
TB = 1024
NT = 3


def _body(x_ref, t_ref, wr_hbm, g1_hbm, b1_hbm, g2_hbm, b2_hbm, wg_hbm,
          we1_hbm, be1_hbm, we2_hbm, be2_hbm, ms_hbm, ti_ref,
          ox_ref, ot_ref, px_ref, pt_ref, aux_ref,
          wr_raw, wbx_ref, wbt_ref, c2_ref, g2_v, b2_v, wg_v, we1_v, be1_v,
          w1g_ref, b1g_ref, s_ref, w2_ref, be2_v, ms_v, sem,
          imp_ref, load_ref):
    i = pl.program_id(0)
    nsteps = pl.num_programs(0)
    ti = ti_ref[0, 0]

    @pl.when(i == 0)
    def _prep():
        cps = [pltpu.make_async_copy(wr_hbm, wr_raw.at[0:RED, :], sem),
               pltpu.make_async_copy(g1_hbm, wr_raw.at[RED:RED + 1, :], sem),
               pltpu.make_async_copy(b1_hbm, wr_raw.at[RED + 1:RED + 2, :],
                                     sem),
               pltpu.make_async_copy(g2_hbm, g2_v, sem),
               pltpu.make_async_copy(b2_hbm, b2_v, sem),
               pltpu.make_async_copy(wg_hbm, wg_v, sem),
               pltpu.make_async_copy(we1_hbm, we1_v, sem),
               pltpu.make_async_copy(be1_hbm, be1_v, sem),
               pltpu.make_async_copy(we2_hbm, w2_ref, sem),
               pltpu.make_async_copy(be2_hbm, be2_v, sem),
               pltpu.make_async_copy(ms_hbm, ms_v, sem)]
        for cp in cps:
            cp.start()
        for cp in cps:
            cp.wait()

        wr = wr_raw[0:RED, :]
        g1 = wr_raw[RED:RED + 1, :]
        wbx_ref[...] = (wr[:, :DIM] * g1[:, :DIM]).astype(jnp.bfloat16)
        wbt_ref[...] = (wr[:, DIM:] * g1[:, DIM:]).astype(jnp.bfloat16)
        c2_ref[...] = jax.lax.dot_general(
            wr_raw[RED + 1:RED + 2, :], wr, (((1,), (1,)), ((), ())),
            preferred_element_type=jnp.float32)
        we1f = jnp.concatenate([we1_v[e] for e in range(E)], axis=1)
        w1g_ref[...] = jnp.concatenate([we1f, wg_v[ti]],
                                       axis=1).astype(jnp.bfloat16)
        b1g_ref[...] = jnp.concatenate(
            [be1_v[e:e + 1, :] for e in range(E)]
            + [jnp.zeros((1, E), jnp.float32)], axis=1)
        col = jax.lax.broadcasted_iota(jnp.int32, (E, EH), 1) // HID
        row = jax.lax.broadcasted_iota(jnp.int32, (E, EH), 0)
        s_ref[...] = (col == row).astype(jnp.bfloat16)

    xb = x_ref[...]
    tb = t_ref[...]

    m = (jnp.sum(xb, axis=1, keepdims=True)
         + jnp.sum(tb, axis=1, keepdims=True)) * (1.0 / (2 * DIM))
    v = (jnp.sum(xb * xb, axis=1, keepdims=True)
         + jnp.sum(tb * tb, axis=1, keepdims=True)) * (1.0 / (2 * DIM)) - m * m
    rs = 1.0 / jnp.sqrt(v + 1e-5)

    xnb = ((xb - m) * rs).astype(jnp.bfloat16)
    tnb = ((tb - m) * rs).astype(jnp.bfloat16)
    u = (jax.lax.dot_general(xnb, wbx_ref[...], (((1,), (1,)), ((), ())),
                             preferred_element_type=jnp.float32)
         + jax.lax.dot_general(tnb, wbt_ref[...], (((1,), (1,)), ((), ())),
                               preferred_element_type=jnp.float32))
    u = u + c2_ref[...]

    m2 = jnp.mean(u, axis=1, keepdims=True)
    uc = u - m2
    v2 = jnp.mean(uc * uc, axis=1, keepdims=True)
    yf = uc * (1.0 / jnp.sqrt(v2 + 1e-5)) * g2_v[...] + b2_v[...]
    zb = yf.astype(jnp.bfloat16)

    r = jnp.dot(zb, w1g_ref[...], preferred_element_type=jnp.float32) + b1g_ref[...]
    h = jnp.maximum(r[:, :EH], 0.0)
    logits = r[:, EH:EH + E]

    pri = (E - jax.lax.broadcasted_iota(jnp.int32, logits.shape, 1)
           ).astype(jnp.float32)
    m1 = jnp.max(logits, axis=1, keepdims=True)
    w1m = jnp.where(logits == m1, pri, 0.0)
    mask1 = w1m == jnp.max(w1m, axis=1, keepdims=True)
    l2 = jnp.where(mask1, -jnp.inf, logits)
    m2g = jnp.max(l2, axis=1, keepdims=True)
    w2m = jnp.where(l2 == m2g, pri, 0.0)
    mask2 = w2m == jnp.max(w2m, axis=1, keepdims=True)
    e2 = jnp.exp(m2g - m1)
    den = 1.0 + e2
    gates = (jnp.where(mask1, 1.0 / den, 0.0)
             + jnp.where(mask2, e2 / den, 0.0))

    gb = gates.astype(jnp.bfloat16)
    gexp = jnp.dot(gb, s_ref[...], preferred_element_type=jnp.float32)
    ghb = (h * gexp).astype(jnp.bfloat16)
    moe = (jnp.dot(ghb, w2_ref[...].astype(jnp.bfloat16),
                   preferred_element_type=jnp.float32)
           + jnp.dot(gb, be2_v[...].astype(jnp.bfloat16),
                     preferred_element_type=jnp.float32))

    px = jax.nn.sigmoid(jnp.mean(moe[:, :RED // 2], axis=1, keepdims=True))
    pt = jax.nn.sigmoid(jnp.mean(moe[:, RED // 2:], axis=1, keepdims=True))

    ox_ref[...] = px * xb + ms_v[pl.ds(2 * ti, 1), :]
    ot_ref[...] = pt * tb + ms_v[pl.ds(2 * ti + 1, 1), :]
    px_ref[...] = jnp.reshape(px, (1, TB // 128, 128))
    pt_ref[...] = jnp.reshape(pt, (1, TB // 128, 128))

    imp_b = jnp.sum(gates, axis=0, keepdims=True)
    load_b = jnp.sum((gates > 0.0).astype(jnp.float32), axis=0, keepdims=True)

    @pl.when(i == 0)
    def _init():
        imp_ref[...] = imp_b
        load_ref[...] = load_b

    @pl.when(i > 0)
    def _acc():
        imp_ref[...] += imp_b
        load_ref[...] += load_b

    @pl.when(i == nsteps - 1)
    def _fin():
        def cv2(a):
            mu = jnp.sum(a, axis=1, keepdims=True) * (1.0 / E)
            var = jnp.sum((a - mu) ** 2, axis=1, keepdims=True) * (1.0 / (E - 1))
            return var / (mu * mu + 1e-10)

        aux_ref[...] = (cv2(imp_ref[...]) + cv2(load_ref[...])) * 1e-2


def kernel(x, t, gamma1, beta1, W_red, gamma2, beta2, w_gate, We1, be1, We2,
           be2, modal_shifts, task_index):
    B, N, C = x.shape
    T = B * N
    xf = x.reshape(T, C)
    tf = t.reshape(T, C)
    ti = jnp.asarray(task_index, jnp.int32).reshape(1, 1)

    grid = (T // TB,)
    tok = lambda i: (i, 0)
    fix = lambda i: (0, 0)
    anyspec = pl.BlockSpec(memory_space=pltpu.MemorySpace.HBM)

    out_x, out_t, pxo, pto, aux = pl.pallas_call(
        _body,
        grid=grid,
        in_specs=[
            pl.BlockSpec((TB, C), tok),
            pl.BlockSpec((TB, C), tok),
            anyspec,
            anyspec,
            anyspec,
            anyspec,
            anyspec,
            anyspec,
            anyspec,
            anyspec,
            anyspec,
            anyspec,
            anyspec,
            pl.BlockSpec(memory_space=pltpu.SMEM),
        ],
        out_specs=[
            pl.BlockSpec((TB, C), tok),
            pl.BlockSpec((TB, C), tok),
            pl.BlockSpec((1, TB // 128, 128), lambda i: (i, 0, 0)),
            pl.BlockSpec((1, TB // 128, 128), lambda i: (i, 0, 0)),
            pl.BlockSpec((1, 1), fix),
        ],
        out_shape=[
            jax.ShapeDtypeStruct((T, C), jnp.float32),
            jax.ShapeDtypeStruct((T, C), jnp.float32),
            jax.ShapeDtypeStruct((T // TB, TB // 128, 128), jnp.float32),
            jax.ShapeDtypeStruct((T // TB, TB // 128, 128), jnp.float32),
            jax.ShapeDtypeStruct((1, 1), jnp.float32),
        ],
        scratch_shapes=[
            pltpu.VMEM((RED + 2, 2 * DIM), jnp.float32),
            pltpu.VMEM((RED, DIM), jnp.bfloat16),
            pltpu.VMEM((RED, DIM), jnp.bfloat16),
            pltpu.VMEM((1, RED), jnp.float32),
            pltpu.VMEM((1, RED), jnp.float32),
            pltpu.VMEM((1, RED), jnp.float32),
            pltpu.VMEM((NT, RED, E), jnp.float32),
            pltpu.VMEM((E, RED, HID), jnp.float32),
            pltpu.VMEM((E, HID), jnp.float32),
            pltpu.VMEM((RED, EH + E), jnp.bfloat16),
            pltpu.VMEM((1, EH + E), jnp.float32),
            pltpu.VMEM((E, EH), jnp.bfloat16),
            pltpu.VMEM((EH, RED), jnp.float32),
            pltpu.VMEM((E, RED), jnp.float32),
            pltpu.VMEM((2 * NT, DIM), jnp.float32),
            pltpu.SemaphoreType.DMA,
            pltpu.VMEM((1, E), jnp.float32),
            pltpu.VMEM((1, E), jnp.float32),
        ],
        compiler_params=pltpu.CompilerParams(
            dimension_semantics=("arbitrary",),
        ),
    )(xf, tf, W_red, gamma1[None, :], beta1[None, :], gamma2[None, :],
      beta2[None, :], w_gate, We1, be1, We2.reshape(EH, RED), be2,
      modal_shifts, ti)

    return (out_x.reshape(B, N, C), out_t.reshape(B, N, C),
            pxo.reshape(B, N, 1), pto.reshape(B, N, 1),
            aux.reshape(()))
